# Initial kernel scaffold; baseline (speedup 1.0000x reference)
import jax
import jax.numpy as jnp
from jax.experimental import pallas as pl


def kernel(grid_coord, feat, serialized_depth, serialized_code, W_embed, b_embed, W_qkv, W_o, W_head, b_head):
    raise NotImplementedError("write your pallas kernel here")


# speedup vs baseline: 1.0866x; 1.0866x over previous
import functools

import jax
import jax.numpy as jnp
from jax.experimental import pallas as pl
from jax.experimental.pallas import tpu as pltpu

N = 65536
D_IN = 6
D = 64
H = 4
DH = D // H
PATCH = 256
C = 19
PB = 8


def _attn_body(g_ref, tg_ref, we_ref, be_ref, wqkv_ref, wo_ref, o_ref, *, has_res):
    scale = 1.0 / (DH ** 0.5)
    x = jnp.dot(g_ref[...], we_ref[...], preferred_element_type=jnp.float32) + be_ref[...]
    if has_res:
        x = x + tg_ref[...]
    for p in range(PB):
        xp = x[p * PATCH:(p + 1) * PATCH]
        qkv = jnp.dot(xp, wqkv_ref[...], preferred_element_type=jnp.float32)
        outs = []
        for h in range(H):
            qh = qkv[:, h * DH:(h + 1) * DH]
            kh = qkv[:, D + h * DH:D + (h + 1) * DH]
            vh = qkv[:, 2 * D + h * DH:2 * D + (h + 1) * DH]
            s = jax.lax.dot_general(qh, kh, (((1,), (1,)), ((), ())),
                                    preferred_element_type=jnp.float32) * scale
            m = jnp.max(s, axis=-1, keepdims=True)
            e = jnp.exp(s - m)
            denom = jnp.sum(e, axis=-1, keepdims=True)
            a = e / denom
            outs.append(jnp.dot(a, vh, preferred_element_type=jnp.float32))
        o = jnp.concatenate(outs, axis=1)
        o_ref[p * PATCH:(p + 1) * PATCH, :] = jnp.dot(
            o, wo_ref[...], preferred_element_type=jnp.float32)


def _attn_pass(g, tg, we, be, wqkv, wo, has_res):
    blk = PB * PATCH
    grid = (N // blk,)
    if tg is None:
        tg = jnp.zeros((8, D), jnp.float32)
        tg_spec = pl.BlockSpec((8, D), lambda i: (0, 0))
    else:
        tg_spec = pl.BlockSpec((blk, D), lambda i: (i, 0))
    return pl.pallas_call(
        functools.partial(_attn_body, has_res=has_res),
        grid=grid,
        in_specs=[
            pl.BlockSpec((blk, D_IN), lambda i: (i, 0)),
            tg_spec,
            pl.BlockSpec((D_IN, D), lambda i: (0, 0)),
            pl.BlockSpec((1, D), lambda i: (0, 0)),
            pl.BlockSpec((D, 3 * D), lambda i: (0, 0)),
            pl.BlockSpec((D, D), lambda i: (0, 0)),
        ],
        out_specs=pl.BlockSpec((blk, D), lambda i: (i, 0)),
        out_shape=jax.ShapeDtypeStruct((N, D), jnp.float32),
    )(g, tg, we, be, wqkv, wo)


def _head_body(feat_ref, t_ref, t2_ref, we_ref, be_ref, wh_ref, bh_ref,
               probs_ref, label_ref):
    x = jnp.dot(feat_ref[...], we_ref[...], preferred_element_type=jnp.float32) + be_ref[...]
    x = x + t_ref[...] + t2_ref[...]
    logits = jnp.dot(x, wh_ref[...], preferred_element_type=jnp.float32) + bh_ref[...]
    m = jnp.max(logits, axis=-1, keepdims=True)
    e = jnp.exp(logits - m)
    probs = e / jnp.sum(e, axis=-1, keepdims=True)
    probs_ref[...] = probs
    label_ref[...] = jnp.argmax(logits, axis=-1).astype(jnp.int32)


def _head(feat, t, t2, we, be, wh, bh):
    blk = 4096
    grid = (N // blk,)
    return pl.pallas_call(
        _head_body,
        grid=grid,
        in_specs=[
            pl.BlockSpec((blk, D_IN), lambda i: (i, 0)),
            pl.BlockSpec((blk, D), lambda i: (i, 0)),
            pl.BlockSpec((blk, D), lambda i: (i, 0)),
            pl.BlockSpec((D_IN, D), lambda i: (0, 0)),
            pl.BlockSpec((1, D), lambda i: (0, 0)),
            pl.BlockSpec((D, C), lambda i: (0, 0)),
            pl.BlockSpec((1, C), lambda i: (0, 0)),
        ],
        out_specs=[
            pl.BlockSpec((blk, C), lambda i: (i, 0)),
            pl.BlockSpec((blk,), lambda i: (i,)),
        ],
        out_shape=[
            jax.ShapeDtypeStruct((N, C), jnp.float32),
            jax.ShapeDtypeStruct((N,), jnp.int32),
        ],
    )(feat, t, t2, we, be, wh, bh)


def kernel(grid_coord, feat, serialized_depth, serialized_code, W_embed,
           b_embed, W_qkv, W_o, W_head, b_head):
    feat = feat.astype(jnp.float32)
    code = serialized_code.astype(jnp.int32)
    order = jnp.argsort(code, axis=1)

    be = b_embed.reshape(1, D)
    bh = b_head.reshape(1, C)

    g0 = jnp.take(feat, order[0], axis=0)
    o0 = _attn_pass(g0, None, W_embed, be, W_qkv[0], W_o[0], has_res=False)
    t = jnp.zeros((N, D), jnp.float32).at[order[0]].set(o0)

    g1 = jnp.take(feat, order[1], axis=0)
    tg1 = jnp.take(t, order[1], axis=0)
    o1 = _attn_pass(g1, tg1, W_embed, be, W_qkv[1], W_o[1], has_res=True)
    t2 = jnp.zeros((N, D), jnp.float32).at[order[1]].set(o1)

    probs, label = _head(feat, t, t2, W_embed, be, W_head, bh)
    return (label, probs)


# speedup vs baseline: 1.3918x; 1.2809x over previous
import functools

import jax
import jax.numpy as jnp
from jax import lax
from jax.experimental import pallas as pl
from jax.experimental.pallas import tpu as pltpu
from jax.experimental.pallas import tpu_sc as plsc

N = 65536
D_IN = 6
DP = 8
D = 64
H = 4
DH = D // H
PATCH = 256
C = 19
PB = 8

NT = 16
CHUNK = N // NT
LCH = CHUNK // 16
RB = 256
NPASS = 4


def _sc_sort_gather(code, featp):
    mesh = plsc.VectorSubcoreMesh(core_axis_name="c", subcore_axis_name="s")

    @functools.partial(
        pl.kernel, mesh=mesh,
        compiler_params=pltpu.CompilerParams(needs_layout_passes=False, use_tc_tiling_on_sc=False),
        out_type=[jax.ShapeDtypeStruct((2, N), jnp.int32),
                  jax.ShapeDtypeStruct((2, N, DP), jnp.float32)],
        scratch_types=[
            pltpu.VMEM((CHUNK,), jnp.int32),
            pltpu.VMEM((CHUNK,), jnp.int32),
            pltpu.VMEM((RB * 16,), jnp.int32),
            pltpu.VMEM((RB,), jnp.int32),
            pltpu.VMEM((RB,), jnp.int32),
            pltpu.VMEM((NT, RB), jnp.int32),
            pltpu.VMEM((CHUNK,), jnp.int32),
            pltpu.VMEM((CHUNK,), jnp.int32),
            pltpu.VMEM((CHUNK,), jnp.int32),
            pltpu.VMEM((CHUNK, DP), jnp.float32),
            pltpu.VMEM_SHARED((N,), jnp.int32),
            pltpu.VMEM_SHARED((N,), jnp.int32),
            pltpu.VMEM_SHARED((N,), jnp.int32),
            pltpu.VMEM_SHARED((N,), jnp.int32),
            pltpu.VMEM_SHARED((NT, RB), jnp.int32),
            pltpu.SemaphoreType.DMA,
        ],
    )
    def k(code_h, feat_h, order_h, g_h, mykeys, myvals, hist, dbase, tilecnt,
          allcnt, keybuf, valbuf, destbuf, gbuf, skA, svA, skB, svB, scnt, sem):
        c = lax.axis_index("c")
        t = lax.axis_index("s")
        lane = lax.iota(jnp.int32, 16)
        ones = jnp.ones((16,), jnp.int32)
        zeros = jnp.zeros((16,), jnp.int32)
        base_t = t * CHUNK

        bufs = [(skB, svB), (skA, svA)]
        for p in range(NPASS):
            shift = 8 * p
            dst_k, dst_v = bufs[p % 2]
            if p == 0:
                pltpu.sync_copy(code_h.at[c, pl.ds(base_t, CHUNK)], mykeys)
            else:
                src_k, src_v = bufs[(p + 1) % 2]
                pltpu.sync_copy(src_k.at[pl.ds(base_t, CHUNK)], mykeys)
                pltpu.sync_copy(src_v.at[pl.ds(base_t, CHUNK)], myvals)

            def zbody(i, _):
                plsc.store_scatter(hist, [i * 16 + lane], zeros)
                return 0
            lax.fori_loop(0, RB, zbody, 0)

            def hbody(i, _):
                kv = plsc.load_gather(mykeys, [lane * LCH + i])
                d = (kv >> shift) & (RB - 1)
                plsc.addupdate_scatter(hist, [d * 16 + lane], ones)
                return 0
            lax.fori_loop(0, LCH, hbody, 0)

            def b1(d, _):
                cell = d * 16 + lane
                row = plsc.load_gather(hist, [cell])
                cs = plsc.cumsum(row)
                plsc.store_scatter(hist, [cell], cs - row)
                plsc.store_scatter(tilecnt, [zeros + d], cs, mask=lane == 15)
                return 0
            lax.fori_loop(0, RB, b1, 0)

            pltpu.sync_copy(tilecnt, scnt.at[t])
            plsc.subcore_barrier()
            pltpu.sync_copy(scnt, allcnt)

            carry = jnp.int32(0)
            for dg in range(RB // 16):
                acc = zeros
                myexcl = zeros
                for tt in range(NT):
                    myexcl = jnp.where(t == tt, acc, myexcl)
                    acc = acc + allcnt[tt, dg * 16:(dg + 1) * 16]
                cs = plsc.cumsum(acc)
                dbase[dg * 16:(dg + 1) * 16] = cs - acc + carry + myexcl
                carry = carry + jnp.sum(acc)

            def cbody(i, _):
                idx = lane * LCH + i
                kv = plsc.load_gather(mykeys, [idx])
                if p == 0:
                    vv = base_t + idx
                else:
                    vv = plsc.load_gather(myvals, [idx])
                d = (kv >> shift) & (RB - 1)
                cell = d * 16 + lane
                cnt = plsc.load_gather(hist, [cell])
                plsc.store_scatter(hist, [cell], cnt + 1)
                db = plsc.load_gather(dbase, [d])
                st = i * 16 + lane
                plsc.store_scatter(keybuf, [st], kv)
                plsc.store_scatter(valbuf, [st], vv)
                plsc.store_scatter(destbuf, [st], db + cnt)
                return 0
            lax.fori_loop(0, LCH, cbody, 0)

            pltpu.sync_copy(keybuf, dst_k.at[destbuf])
            pltpu.sync_copy(valbuf, dst_v.at[destbuf])
            plsc.subcore_barrier()

        _, fin_v = bufs[(NPASS - 1) % 2]
        pltpu.sync_copy(fin_v.at[pl.ds(base_t, CHUNK)], myvals)
        pltpu.sync_copy(myvals, order_h.at[c, pl.ds(base_t, CHUNK)])
        pltpu.async_copy(feat_h.at[myvals], gbuf, sem).wait()
        pltpu.sync_copy(gbuf, g_h.at[c, pl.ds(base_t, CHUNK), :])

    return k(code, featp)


RCH = N // 32
SUB = 1024


def _sc_scatter_rows(o, order, row):
    mesh = plsc.VectorSubcoreMesh(core_axis_name="c", subcore_axis_name="s")

    @functools.partial(
        pl.kernel, mesh=mesh,
        compiler_params=pltpu.CompilerParams(needs_layout_passes=False, use_tc_tiling_on_sc=False),
        out_type=jax.ShapeDtypeStruct((N, D), jnp.float32),
        scratch_types=[
            pltpu.VMEM((2, SUB), jnp.int32),
            pltpu.VMEM((SUB, D), jnp.float32),
            pltpu.SemaphoreType.DMA,
        ],
    )
    def k(o_h, ord_h, t_h, idxbuf, obuf, sem):
        c = lax.axis_index("c")
        s = lax.axis_index("s")
        wid = s * 2 + c
        for j in range(RCH // SUB):
            base = wid * RCH + j * SUB
            pltpu.sync_copy(ord_h.at[row, pl.ds(base, SUB)], idxbuf.at[j])
            pltpu.sync_copy(o_h.at[pl.ds(base, SUB), :], obuf)
            pltpu.async_copy(obuf, t_h.at[idxbuf.at[j]], sem).wait()

    return k(o, order)


def _sc_gather_rows(t, order, row):
    mesh = plsc.VectorSubcoreMesh(core_axis_name="c", subcore_axis_name="s")

    @functools.partial(
        pl.kernel, mesh=mesh,
        compiler_params=pltpu.CompilerParams(needs_layout_passes=False, use_tc_tiling_on_sc=False),
        out_type=jax.ShapeDtypeStruct((N, D), jnp.float32),
        scratch_types=[
            pltpu.VMEM((2, SUB), jnp.int32),
            pltpu.VMEM((SUB, D), jnp.float32),
            pltpu.SemaphoreType.DMA,
        ],
    )
    def k(t_h, ord_h, out_h, idxbuf, obuf, sem):
        c = lax.axis_index("c")
        s = lax.axis_index("s")
        wid = s * 2 + c
        for j in range(RCH // SUB):
            base = wid * RCH + j * SUB
            pltpu.sync_copy(ord_h.at[row, pl.ds(base, SUB)], idxbuf.at[j])
            pltpu.async_copy(t_h.at[idxbuf.at[j]], obuf, sem).wait()
            pltpu.sync_copy(obuf, out_h.at[pl.ds(base, SUB), :])

    return k(t, order)


def _attn_body(g_ref, tg_ref, we_ref, be_ref, wqkv_ref, wo_ref, o_ref, *, has_res):
    scale = 1.0 / (DH ** 0.5)
    x = jnp.dot(g_ref[...], we_ref[...], preferred_element_type=jnp.float32) + be_ref[...]
    if has_res:
        x = x + tg_ref[...]
    for p in range(PB):
        xp = x[p * PATCH:(p + 1) * PATCH]
        qkv = jnp.dot(xp, wqkv_ref[...], preferred_element_type=jnp.float32)
        outs = []
        for h in range(H):
            qh = qkv[:, h * DH:(h + 1) * DH]
            kh = qkv[:, D + h * DH:D + (h + 1) * DH]
            vh = qkv[:, 2 * D + h * DH:2 * D + (h + 1) * DH]
            s = jax.lax.dot_general(qh, kh, (((1,), (1,)), ((), ())),
                                    preferred_element_type=jnp.float32) * scale
            m = jnp.max(s, axis=-1, keepdims=True)
            e = jnp.exp(s - m)
            denom = jnp.sum(e, axis=-1, keepdims=True)
            a = e / denom
            outs.append(jnp.dot(a, vh, preferred_element_type=jnp.float32))
        o = jnp.concatenate(outs, axis=1)
        o_ref[p * PATCH:(p + 1) * PATCH, :] = jnp.dot(
            o, wo_ref[...], preferred_element_type=jnp.float32)


def _attn_pass(g, tg, we, be, wqkv, wo, has_res):
    blk = PB * PATCH
    grid = (N // blk,)
    if tg is None:
        tg = jnp.zeros((8, D), jnp.float32)
        tg_spec = pl.BlockSpec((8, D), lambda i: (0, 0))
    else:
        tg_spec = pl.BlockSpec((blk, D), lambda i: (i, 0))
    return pl.pallas_call(
        functools.partial(_attn_body, has_res=has_res),
        grid=grid,
        in_specs=[
            pl.BlockSpec((blk, DP), lambda i: (i, 0)),
            tg_spec,
            pl.BlockSpec((DP, D), lambda i: (0, 0)),
            pl.BlockSpec((1, D), lambda i: (0, 0)),
            pl.BlockSpec((D, 3 * D), lambda i: (0, 0)),
            pl.BlockSpec((D, D), lambda i: (0, 0)),
        ],
        out_specs=pl.BlockSpec((blk, D), lambda i: (i, 0)),
        out_shape=jax.ShapeDtypeStruct((N, D), jnp.float32),
    )(g, tg, we, be, wqkv, wo)


def _head_body(feat_ref, t_ref, t2_ref, we_ref, be_ref, wh_ref, bh_ref,
               probs_ref, label_ref):
    x = jnp.dot(feat_ref[...], we_ref[...], preferred_element_type=jnp.float32) + be_ref[...]
    x = x + t_ref[...] + t2_ref[...]
    logits = jnp.dot(x, wh_ref[...], preferred_element_type=jnp.float32) + bh_ref[...]
    m = jnp.max(logits, axis=-1, keepdims=True)
    e = jnp.exp(logits - m)
    probs = e / jnp.sum(e, axis=-1, keepdims=True)
    probs_ref[...] = probs
    label_ref[...] = jnp.argmax(logits, axis=-1).astype(jnp.int32)


def _head(feat, t, t2, we, be, wh, bh):
    blk = 4096
    grid = (N // blk,)
    return pl.pallas_call(
        _head_body,
        grid=grid,
        in_specs=[
            pl.BlockSpec((blk, DP), lambda i: (i, 0)),
            pl.BlockSpec((blk, D), lambda i: (i, 0)),
            pl.BlockSpec((blk, D), lambda i: (i, 0)),
            pl.BlockSpec((DP, D), lambda i: (0, 0)),
            pl.BlockSpec((1, D), lambda i: (0, 0)),
            pl.BlockSpec((D, C), lambda i: (0, 0)),
            pl.BlockSpec((1, C), lambda i: (0, 0)),
        ],
        out_specs=[
            pl.BlockSpec((blk, C), lambda i: (i, 0)),
            pl.BlockSpec((blk,), lambda i: (i,)),
        ],
        out_shape=[
            jax.ShapeDtypeStruct((N, C), jnp.float32),
            jax.ShapeDtypeStruct((N,), jnp.int32),
        ],
    )(feat, t, t2, we, be, wh, bh)


def kernel(grid_coord, feat, serialized_depth, serialized_code, W_embed,
           b_embed, W_qkv, W_o, W_head, b_head):
    feat = feat.astype(jnp.float32)
    code = serialized_code.astype(jnp.int32)
    featp = jnp.pad(feat, ((0, 0), (0, DP - D_IN)))
    wep = jnp.pad(W_embed, ((0, DP - D_IN), (0, 0)))
    be = b_embed.reshape(1, D)
    bh = b_head.reshape(1, C)

    order, g = _sc_sort_gather(code, featp)

    o0 = _attn_pass(g[0], None, wep, be, W_qkv[0], W_o[0], has_res=False)
    t = _sc_scatter_rows(o0, order, 0)
    tg1 = _sc_gather_rows(t, order, 1)
    o1 = _attn_pass(g[1], tg1, wep, be, W_qkv[1], W_o[1], has_res=True)
    t2 = _sc_scatter_rows(o1, order, 1)

    probs, label = _head(featp, t, t2, wep, be, W_head, bh)
    return (label, probs)


# speedup vs baseline: 1.6910x; 1.2149x over previous
import functools

import jax
import jax.numpy as jnp
from jax import lax
from jax.experimental import pallas as pl
from jax.experimental.pallas import tpu as pltpu
from jax.experimental.pallas import tpu_sc as plsc

N = 65536
D_IN = 6
DP = 8
D = 64
H = 4
DH = D // H
PATCH = 256
C = 19
PB = 8

NT = 16
CHUNK = N // NT
LCH = CHUNK // 16
RB = 256
NPASS = 4


def _sc_sort_gather(code, featp):
    mesh = plsc.VectorSubcoreMesh(core_axis_name="c", subcore_axis_name="s")

    @functools.partial(
        pl.kernel, mesh=mesh,
        compiler_params=pltpu.CompilerParams(needs_layout_passes=False, use_tc_tiling_on_sc=False),
        out_type=[jax.ShapeDtypeStruct((2, N), jnp.int32),
                  jax.ShapeDtypeStruct((2, N, DP), jnp.float32)],
        scratch_types=[
            pltpu.VMEM((CHUNK,), jnp.int32),
            pltpu.VMEM((CHUNK,), jnp.int32),
            pltpu.VMEM((RB * 16,), jnp.int32),
            pltpu.VMEM((RB,), jnp.int32),
            pltpu.VMEM((RB,), jnp.int32),
            pltpu.VMEM((NT, RB), jnp.int32),
            pltpu.VMEM((CHUNK,), jnp.int32),
            pltpu.VMEM((CHUNK,), jnp.int32),
            pltpu.VMEM((CHUNK,), jnp.int32),
            pltpu.VMEM((CHUNK, DP), jnp.float32),
            pltpu.VMEM_SHARED((N,), jnp.int32),
            pltpu.VMEM_SHARED((N,), jnp.int32),
            pltpu.VMEM_SHARED((N,), jnp.int32),
            pltpu.VMEM_SHARED((N,), jnp.int32),
            pltpu.VMEM_SHARED((NT, RB), jnp.int32),
            pltpu.SemaphoreType.DMA,
        ],
    )
    def k(code_h, feat_h, order_h, g_h, mykeys, myvals, hist, dbase, tilecnt,
          allcnt, keybuf, valbuf, destbuf, gbuf, skA, svA, skB, svB, scnt, sem):
        c = lax.axis_index("c")
        t = lax.axis_index("s")
        lane = lax.iota(jnp.int32, 16)
        ones = jnp.ones((16,), jnp.int32)
        zeros = jnp.zeros((16,), jnp.int32)
        base_t = t * CHUNK

        bufs = [(skB, svB), (skA, svA)]
        for p in range(NPASS):
            shift = 8 * p
            dst_k, dst_v = bufs[p % 2]
            if p == 0:
                pltpu.sync_copy(code_h.at[c, pl.ds(base_t, CHUNK)], mykeys)
            else:
                src_k, src_v = bufs[(p + 1) % 2]
                pltpu.sync_copy(src_k.at[pl.ds(base_t, CHUNK)], mykeys)
                pltpu.sync_copy(src_v.at[pl.ds(base_t, CHUNK)], myvals)

            def zbody(i, _):
                plsc.store_scatter(hist, [i * 16 + lane], zeros)
                return 0
            lax.fori_loop(0, RB, zbody, 0)

            def hbody(i, _):
                kv = plsc.load_gather(mykeys, [lane * LCH + i])
                d = (kv >> shift) & (RB - 1)
                plsc.addupdate_scatter(hist, [d * 16 + lane], ones)
                return 0
            lax.fori_loop(0, LCH, hbody, 0)

            def b1(d, _):
                cell = d * 16 + lane
                row = plsc.load_gather(hist, [cell])
                cs = plsc.cumsum(row)
                plsc.store_scatter(hist, [cell], cs - row)
                plsc.store_scatter(tilecnt, [zeros + d], cs, mask=lane == 15)
                return 0
            lax.fori_loop(0, RB, b1, 0)

            pltpu.sync_copy(tilecnt, scnt.at[t])
            plsc.subcore_barrier()
            pltpu.sync_copy(scnt, allcnt)

            carry = jnp.int32(0)
            for dg in range(RB // 16):
                acc = zeros
                myexcl = zeros
                for tt in range(NT):
                    myexcl = jnp.where(t == tt, acc, myexcl)
                    acc = acc + allcnt[tt, dg * 16:(dg + 1) * 16]
                cs = plsc.cumsum(acc)
                dbase[dg * 16:(dg + 1) * 16] = cs - acc + carry + myexcl
                carry = carry + jnp.sum(acc)

            def cbody(i, _):
                idx = lane * LCH + i
                kv = plsc.load_gather(mykeys, [idx])
                if p == 0:
                    vv = base_t + idx
                else:
                    vv = plsc.load_gather(myvals, [idx])
                d = (kv >> shift) & (RB - 1)
                cell = d * 16 + lane
                cnt = plsc.load_gather(hist, [cell])
                plsc.store_scatter(hist, [cell], cnt + 1)
                db = plsc.load_gather(dbase, [d])
                st = i * 16 + lane
                plsc.store_scatter(keybuf, [st], kv)
                plsc.store_scatter(valbuf, [st], vv)
                plsc.store_scatter(destbuf, [st], db + cnt)
                return 0
            lax.fori_loop(0, LCH, cbody, 0)

            pltpu.sync_copy(keybuf, dst_k.at[destbuf])
            pltpu.sync_copy(valbuf, dst_v.at[destbuf])
            plsc.subcore_barrier()

        _, fin_v = bufs[(NPASS - 1) % 2]
        pltpu.sync_copy(fin_v.at[pl.ds(base_t, CHUNK)], myvals)
        pltpu.sync_copy(myvals, order_h.at[c, pl.ds(base_t, CHUNK)])
        pltpu.async_copy(feat_h.at[myvals], gbuf, sem).wait()
        pltpu.sync_copy(gbuf, g_h.at[c, pl.ds(base_t, CHUNK), :])

    return k(code, featp)


RCH = N // 32
SUB = 1024


def _sc_scatter_rows(o, order, row):
    mesh = plsc.VectorSubcoreMesh(core_axis_name="c", subcore_axis_name="s")

    @functools.partial(
        pl.kernel, mesh=mesh,
        compiler_params=pltpu.CompilerParams(needs_layout_passes=False, use_tc_tiling_on_sc=False),
        out_type=jax.ShapeDtypeStruct((N, D), jnp.float32),
        scratch_types=[
            pltpu.VMEM((2, SUB), jnp.int32),
            pltpu.VMEM((SUB, D), jnp.float32),
            pltpu.SemaphoreType.DMA,
        ],
    )
    def k(o_h, ord_h, t_h, idxbuf, obuf, sem):
        c = lax.axis_index("c")
        s = lax.axis_index("s")
        wid = s * 2 + c
        for j in range(RCH // SUB):
            base = wid * RCH + j * SUB
            pltpu.sync_copy(ord_h.at[row, pl.ds(base, SUB)], idxbuf.at[j])
            pltpu.sync_copy(o_h.at[pl.ds(base, SUB), :], obuf)
            pltpu.async_copy(obuf, t_h.at[idxbuf.at[j]], sem).wait()

    return k(o, order)


def _sc_gather_rows(t, order, row):
    mesh = plsc.VectorSubcoreMesh(core_axis_name="c", subcore_axis_name="s")

    @functools.partial(
        pl.kernel, mesh=mesh,
        compiler_params=pltpu.CompilerParams(needs_layout_passes=False, use_tc_tiling_on_sc=False),
        out_type=jax.ShapeDtypeStruct((N, D), jnp.float32),
        scratch_types=[
            pltpu.VMEM((2, SUB), jnp.int32),
            pltpu.VMEM((SUB, D), jnp.float32),
            pltpu.SemaphoreType.DMA,
        ],
    )
    def k(t_h, ord_h, out_h, idxbuf, obuf, sem):
        c = lax.axis_index("c")
        s = lax.axis_index("s")
        wid = s * 2 + c
        for j in range(RCH // SUB):
            base = wid * RCH + j * SUB
            pltpu.sync_copy(ord_h.at[row, pl.ds(base, SUB)], idxbuf.at[j])
            pltpu.async_copy(t_h.at[idxbuf.at[j]], obuf, sem).wait()
            pltpu.sync_copy(obuf, out_h.at[pl.ds(base, SUB), :])

    return k(t, order)


def _attn_body(g_ref, tg_ref, we_ref, be_ref, wqkv_ref, wo_ref, o_ref, *, has_res):
    x = jnp.dot(g_ref[...], we_ref[...], preferred_element_type=jnp.float32) + be_ref[...]
    if has_res:
        x = x + tg_ref[...]
    for p in range(PB):
        xp = x[p * PATCH:(p + 1) * PATCH]
        qkv = jnp.dot(xp, wqkv_ref[...], preferred_element_type=jnp.float32)
        outs = []
        for h in range(H):
            qh = qkv[:, h * DH:(h + 1) * DH]
            kh = qkv[:, D + h * DH:D + (h + 1) * DH]
            vh = qkv[:, 2 * D + h * DH:2 * D + (h + 1) * DH]
            s = jax.lax.dot_general(qh, kh, (((1,), (1,)), ((), ())),
                                    preferred_element_type=jnp.float32)
            e = jnp.exp(s)
            denom = jnp.sum(e, axis=-1, keepdims=True)
            ev = jnp.dot(e, vh, preferred_element_type=jnp.float32)
            outs.append(ev * (1.0 / denom))
        o = jnp.concatenate(outs, axis=1)
        o_ref[p * PATCH:(p + 1) * PATCH, :] = jnp.dot(
            o, wo_ref[...], preferred_element_type=jnp.float32)


def _attn_pass(g, tg, we, be, wqkv, wo, has_res):
    blk = PB * PATCH
    grid = (N // blk,)
    if tg is None:
        tg = jnp.zeros((8, D), jnp.float32)
        tg_spec = pl.BlockSpec((8, D), lambda i: (0, 0))
    else:
        tg_spec = pl.BlockSpec((blk, D), lambda i: (i, 0))
    return pl.pallas_call(
        functools.partial(_attn_body, has_res=has_res),
        grid=grid,
        in_specs=[
            pl.BlockSpec((blk, DP), lambda i: (i, 0)),
            tg_spec,
            pl.BlockSpec((DP, D), lambda i: (0, 0)),
            pl.BlockSpec((1, D), lambda i: (0, 0)),
            pl.BlockSpec((D, 3 * D), lambda i: (0, 0)),
            pl.BlockSpec((D, D), lambda i: (0, 0)),
        ],
        out_specs=pl.BlockSpec((blk, D), lambda i: (i, 0)),
        out_shape=jax.ShapeDtypeStruct((N, D), jnp.float32),
    )(g, tg, we, be, wqkv, wo)


def _head_body(feat_ref, t_ref, t2_ref, we_ref, be_ref, wh_ref, bh_ref,
               probs_ref, label_ref):
    x = jnp.dot(feat_ref[...], we_ref[...], preferred_element_type=jnp.float32) + be_ref[...]
    x = x + t_ref[...] + t2_ref[...]
    logits = jnp.dot(x, wh_ref[...], preferred_element_type=jnp.float32) + bh_ref[...]
    m = jnp.max(logits, axis=-1, keepdims=True)
    e = jnp.exp(logits - m)
    probs = e / jnp.sum(e, axis=-1, keepdims=True)
    probs_ref[...] = probs
    label_ref[...] = jnp.argmax(logits, axis=-1).astype(jnp.int32)


def _head(feat, t, t2, we, be, wh, bh):
    blk = 4096
    grid = (N // blk,)
    return pl.pallas_call(
        _head_body,
        grid=grid,
        in_specs=[
            pl.BlockSpec((blk, DP), lambda i: (i, 0)),
            pl.BlockSpec((blk, D), lambda i: (i, 0)),
            pl.BlockSpec((blk, D), lambda i: (i, 0)),
            pl.BlockSpec((DP, D), lambda i: (0, 0)),
            pl.BlockSpec((1, D), lambda i: (0, 0)),
            pl.BlockSpec((D, C), lambda i: (0, 0)),
            pl.BlockSpec((1, C), lambda i: (0, 0)),
        ],
        out_specs=[
            pl.BlockSpec((blk, C), lambda i: (i, 0)),
            pl.BlockSpec((blk,), lambda i: (i,)),
        ],
        out_shape=[
            jax.ShapeDtypeStruct((N, C), jnp.float32),
            jax.ShapeDtypeStruct((N,), jnp.int32),
        ],
    )(feat, t, t2, we, be, wh, bh)


def kernel(grid_coord, feat, serialized_depth, serialized_code, W_embed,
           b_embed, W_qkv, W_o, W_head, b_head):
    feat = feat.astype(jnp.float32)
    code = serialized_code.astype(jnp.int32)
    featp = jnp.pad(feat, ((0, 0), (0, DP - D_IN)))
    wep = jnp.pad(W_embed, ((0, DP - D_IN), (0, 0)))
    be = b_embed.reshape(1, D)
    bh = b_head.reshape(1, C)

    W_qkv = W_qkv.at[:, :, :D].multiply(0.25)

    order, g = _sc_sort_gather(code, featp)

    o0 = _attn_pass(g[0], None, wep, be, W_qkv[0], W_o[0], has_res=False)
    t = _sc_scatter_rows(o0, order, 0)
    tg1 = _sc_gather_rows(t, order, 1)
    o1 = _attn_pass(g[1], tg1, wep, be, W_qkv[1], W_o[1], has_res=True)
    t2 = _sc_scatter_rows(o1, order, 1)

    probs, label = _head(featp, t, t2, wep, be, W_head, bh)
    return (label, probs)


# speedup vs baseline: 2.5042x; 1.4809x over previous
import functools

import jax
import jax.numpy as jnp
from jax import lax
from jax.experimental import pallas as pl
from jax.experimental.pallas import tpu as pltpu
from jax.experimental.pallas import tpu_sc as plsc

N = 65536
D_IN = 6
DP = 8
D = 64
H = 4
DH = D // H
PATCH = 256
C = 19
PB = 8

NT = 16
CHUNK = N // NT
LCH = CHUNK // 16
RB = 256
NPASS = 4


def _sc_sort_gather(code, featp):
    mesh = plsc.VectorSubcoreMesh(core_axis_name="c", subcore_axis_name="s")

    @functools.partial(
        pl.kernel, mesh=mesh,
        compiler_params=pltpu.CompilerParams(needs_layout_passes=False, use_tc_tiling_on_sc=False),
        out_type=[jax.ShapeDtypeStruct((2, N), jnp.int32),
                  jax.ShapeDtypeStruct((2, N, DP), jnp.float32)],
        scratch_types=[
            pltpu.VMEM((CHUNK,), jnp.int32),
            pltpu.VMEM((CHUNK,), jnp.int32),
            pltpu.VMEM((RB * 16,), jnp.int32),
            pltpu.VMEM((RB,), jnp.int32),
            pltpu.VMEM((RB,), jnp.int32),
            pltpu.VMEM((NT, RB), jnp.int32),
            pltpu.VMEM((CHUNK,), jnp.int32),
            pltpu.VMEM((CHUNK,), jnp.int32),
            pltpu.VMEM((CHUNK,), jnp.int32),
            pltpu.VMEM((CHUNK, DP), jnp.float32),
            pltpu.VMEM_SHARED((N,), jnp.int32),
            pltpu.VMEM_SHARED((N,), jnp.int32),
            pltpu.VMEM_SHARED((N,), jnp.int32),
            pltpu.VMEM_SHARED((N,), jnp.int32),
            pltpu.VMEM_SHARED((NT, RB), jnp.int32),
            pltpu.SemaphoreType.DMA,
        ],
    )
    def k(code_h, feat_h, order_h, g_h, mykeys, myvals, hist, dbase, tilecnt,
          allcnt, keybuf, valbuf, destbuf, gbuf, skA, svA, skB, svB, scnt, sem):
        c = lax.axis_index("c")
        t = lax.axis_index("s")
        lane = lax.iota(jnp.int32, 16)
        ones = jnp.ones((16,), jnp.int32)
        zeros = jnp.zeros((16,), jnp.int32)
        base_t = t * CHUNK

        bufs = [(skB, svB), (skA, svA)]
        for p in range(NPASS):
            shift = 8 * p
            dst_k, dst_v = bufs[p % 2]
            if p == 0:
                pltpu.sync_copy(code_h.at[c, pl.ds(base_t, CHUNK)], mykeys)
            else:
                src_k, src_v = bufs[(p + 1) % 2]
                pltpu.sync_copy(src_k.at[pl.ds(base_t, CHUNK)], mykeys)
                pltpu.sync_copy(src_v.at[pl.ds(base_t, CHUNK)], myvals)

            def zbody(i, _):
                plsc.store_scatter(hist, [i * 16 + lane], zeros)
                return 0
            lax.fori_loop(0, RB, zbody, 0)

            def hbody(i, _):
                kv = plsc.load_gather(mykeys, [lane * LCH + i])
                d = (kv >> shift) & (RB - 1)
                plsc.addupdate_scatter(hist, [d * 16 + lane], ones)
                return 0
            lax.fori_loop(0, LCH, hbody, 0)

            def b1(d, _):
                cell = d * 16 + lane
                row = plsc.load_gather(hist, [cell])
                cs = plsc.cumsum(row)
                plsc.store_scatter(hist, [cell], cs - row)
                plsc.store_scatter(tilecnt, [zeros + d], cs, mask=lane == 15)
                return 0
            lax.fori_loop(0, RB, b1, 0)

            pltpu.sync_copy(tilecnt, scnt.at[t])
            plsc.subcore_barrier()
            pltpu.sync_copy(scnt, allcnt)

            carry = jnp.int32(0)
            for dg in range(RB // 16):
                acc = zeros
                myexcl = zeros
                for tt in range(NT):
                    myexcl = jnp.where(t == tt, acc, myexcl)
                    acc = acc + allcnt[tt, dg * 16:(dg + 1) * 16]
                cs = plsc.cumsum(acc)
                dbase[dg * 16:(dg + 1) * 16] = cs - acc + carry + myexcl
                carry = carry + jnp.sum(acc)

            def cbody(i, _):
                idx = lane * LCH + i
                kv = plsc.load_gather(mykeys, [idx])
                if p == 0:
                    vv = base_t + idx
                else:
                    vv = plsc.load_gather(myvals, [idx])
                d = (kv >> shift) & (RB - 1)
                cell = d * 16 + lane
                cnt = plsc.load_gather(hist, [cell])
                plsc.store_scatter(hist, [cell], cnt + 1)
                db = plsc.load_gather(dbase, [d])
                st = i * 16 + lane
                plsc.store_scatter(keybuf, [st], kv)
                plsc.store_scatter(valbuf, [st], vv)
                plsc.store_scatter(destbuf, [st], db + cnt)
                return 0
            lax.fori_loop(0, LCH, cbody, 0)

            pltpu.sync_copy(keybuf, dst_k.at[destbuf])
            pltpu.sync_copy(valbuf, dst_v.at[destbuf])
            plsc.subcore_barrier()

        _, fin_v = bufs[(NPASS - 1) % 2]
        pltpu.sync_copy(fin_v.at[pl.ds(base_t, CHUNK)], myvals)
        pltpu.sync_copy(myvals, order_h.at[c, pl.ds(base_t, CHUNK)])
        pltpu.async_copy(feat_h.at[myvals], gbuf, sem).wait()
        pltpu.sync_copy(gbuf, g_h.at[c, pl.ds(base_t, CHUNK), :])

    return k(code, featp)


RCH = N // 32
SUB = 1024


def _sc_scatter_rows(o, order, row):
    mesh = plsc.VectorSubcoreMesh(core_axis_name="c", subcore_axis_name="s")

    @functools.partial(
        pl.kernel, mesh=mesh,
        compiler_params=pltpu.CompilerParams(needs_layout_passes=False, use_tc_tiling_on_sc=False),
        out_type=jax.ShapeDtypeStruct((N, D), jnp.float32),
        scratch_types=[
            pltpu.VMEM((2, SUB), jnp.int32),
            pltpu.VMEM((SUB, D), jnp.float32),
            pltpu.SemaphoreType.DMA,
        ],
    )
    def k(o_h, ord_h, t_h, idxbuf, obuf, sem):
        c = lax.axis_index("c")
        s = lax.axis_index("s")
        wid = s * 2 + c
        for j in range(RCH // SUB):
            base = wid * RCH + j * SUB
            pltpu.sync_copy(ord_h.at[row, pl.ds(base, SUB)], idxbuf.at[j])
            pltpu.sync_copy(o_h.at[pl.ds(base, SUB), :], obuf)
            pltpu.async_copy(obuf, t_h.at[idxbuf.at[j]], sem).wait()

    return k(o, order)


def _sc_gather_rows(t, order, row):
    mesh = plsc.VectorSubcoreMesh(core_axis_name="c", subcore_axis_name="s")

    @functools.partial(
        pl.kernel, mesh=mesh,
        compiler_params=pltpu.CompilerParams(needs_layout_passes=False, use_tc_tiling_on_sc=False),
        out_type=jax.ShapeDtypeStruct((N, D), jnp.float32),
        scratch_types=[
            pltpu.VMEM((2, SUB), jnp.int32),
            pltpu.VMEM((SUB, D), jnp.float32),
            pltpu.SemaphoreType.DMA,
        ],
    )
    def k(t_h, ord_h, out_h, idxbuf, obuf, sem):
        c = lax.axis_index("c")
        s = lax.axis_index("s")
        wid = s * 2 + c
        for j in range(RCH // SUB):
            base = wid * RCH + j * SUB
            pltpu.sync_copy(ord_h.at[row, pl.ds(base, SUB)], idxbuf.at[j])
            pltpu.async_copy(t_h.at[idxbuf.at[j]], obuf, sem).wait()
            pltpu.sync_copy(obuf, out_h.at[pl.ds(base, SUB), :])

    return k(t, order)


def _attn_body(g_ref, tg_ref, we_ref, be_ref, wqkv_ref, wo_ref, o_ref, *, has_res):
    x = jnp.dot(g_ref[...], we_ref[...], preferred_element_type=jnp.float32) + be_ref[...]
    if has_res:
        x = x + tg_ref[...]
    qkvs = [jnp.dot(x[p * PATCH:(p + 1) * PATCH], wqkv_ref[...],
                    preferred_element_type=jnp.float32) for p in range(PB)]
    ss = [jax.lax.dot_general(
              qkvs[p][:, h * DH:(h + 1) * DH],
              qkvs[p][:, D + h * DH:D + (h + 1) * DH],
              (((1,), (1,)), ((), ())), preferred_element_type=jnp.float32)
          for p in range(PB) for h in range(H)]
    ms = [jnp.max(s, axis=-1, keepdims=True) for s in ss]
    es = [jnp.exp(s - m) for s, m in zip(ss, ms)]
    dens = [jnp.sum(e, axis=-1, keepdims=True) for e in es]
    aas = [e / den for e, den in zip(es, dens)]
    avs = [jnp.dot(aas[p * H + h],
                   qkvs[p][:, 2 * D + h * DH:2 * D + (h + 1) * DH],
                   preferred_element_type=jnp.float32)
           for p in range(PB) for h in range(H)]
    for p in range(PB):
        o = jnp.concatenate(avs[p * H:(p + 1) * H], axis=1)
        o_ref[p * PATCH:(p + 1) * PATCH, :] = jnp.dot(
            o, wo_ref[...], preferred_element_type=jnp.float32)


def _attn_pass(g, tg, we, be, wqkv, wo, has_res):
    blk = PB * PATCH
    grid = (N // blk,)
    if tg is None:
        tg = jnp.zeros((8, D), jnp.float32)
        tg_spec = pl.BlockSpec((8, D), lambda i: (0, 0))
    else:
        tg_spec = pl.BlockSpec((blk, D), lambda i: (i, 0))
    return pl.pallas_call(
        functools.partial(_attn_body, has_res=has_res),
        grid=grid,
        in_specs=[
            pl.BlockSpec((blk, DP), lambda i: (i, 0)),
            tg_spec,
            pl.BlockSpec((DP, D), lambda i: (0, 0)),
            pl.BlockSpec((1, D), lambda i: (0, 0)),
            pl.BlockSpec((D, 3 * D), lambda i: (0, 0)),
            pl.BlockSpec((D, D), lambda i: (0, 0)),
        ],
        out_specs=pl.BlockSpec((blk, D), lambda i: (i, 0)),
        out_shape=jax.ShapeDtypeStruct((N, D), jnp.float32),
    )(g, tg, we, be, wqkv, wo)


def _head_body(feat_ref, t_ref, t2_ref, we_ref, be_ref, wh_ref, bh_ref,
               probs_ref, label_ref):
    x = jnp.dot(feat_ref[...], we_ref[...], preferred_element_type=jnp.float32) + be_ref[...]
    x = x + t_ref[...] + t2_ref[...]
    logits = jnp.dot(x, wh_ref[...], preferred_element_type=jnp.float32) + bh_ref[...]
    m = jnp.max(logits, axis=-1, keepdims=True)
    e = jnp.exp(logits - m)
    probs = e / jnp.sum(e, axis=-1, keepdims=True)
    probs_ref[...] = probs
    label_ref[...] = jnp.argmax(logits, axis=-1).astype(jnp.int32)


def _head(feat, t, t2, we, be, wh, bh):
    blk = 4096
    grid = (N // blk,)
    return pl.pallas_call(
        _head_body,
        grid=grid,
        in_specs=[
            pl.BlockSpec((blk, DP), lambda i: (i, 0)),
            pl.BlockSpec((blk, D), lambda i: (i, 0)),
            pl.BlockSpec((blk, D), lambda i: (i, 0)),
            pl.BlockSpec((DP, D), lambda i: (0, 0)),
            pl.BlockSpec((1, D), lambda i: (0, 0)),
            pl.BlockSpec((D, C), lambda i: (0, 0)),
            pl.BlockSpec((1, C), lambda i: (0, 0)),
        ],
        out_specs=[
            pl.BlockSpec((blk, C), lambda i: (i, 0)),
            pl.BlockSpec((blk,), lambda i: (i,)),
        ],
        out_shape=[
            jax.ShapeDtypeStruct((N, C), jnp.float32),
            jax.ShapeDtypeStruct((N,), jnp.int32),
        ],
    )(feat, t, t2, we, be, wh, bh)


def kernel(grid_coord, feat, serialized_depth, serialized_code, W_embed,
           b_embed, W_qkv, W_o, W_head, b_head):
    feat = feat.astype(jnp.float32)
    code = serialized_code.astype(jnp.int32)
    featp = jnp.pad(feat, ((0, 0), (0, DP - D_IN)))
    wep = jnp.pad(W_embed, ((0, DP - D_IN), (0, 0)))
    be = b_embed.reshape(1, D)
    bh = b_head.reshape(1, C)

    W_qkv = W_qkv.at[:, :, :D].multiply(0.25)

    order, g = _sc_sort_gather(code, featp)

    o0 = _attn_pass(g[0], None, wep, be, W_qkv[0], W_o[0], has_res=False)
    t = _sc_scatter_rows(o0, order, 0)
    tg1 = _sc_gather_rows(t, order, 1)
    o1 = _attn_pass(g[1], tg1, wep, be, W_qkv[1], W_o[1], has_res=True)
    t2 = _sc_scatter_rows(o1, order, 1)

    probs, label = _head(featp, t, t2, wep, be, W_head, bh)
    return (label, probs)


# speedup vs baseline: 3.2294x; 1.2896x over previous
import functools

import jax
import jax.numpy as jnp
from jax import lax
from jax.experimental import pallas as pl
from jax.experimental.pallas import tpu as pltpu
from jax.experimental.pallas import tpu_sc as plsc

N = 65536
D_IN = 6
D = 64
DR = 128
H = 4
DH = D // H
PATCH = 256
C = 19
PB = 8

NT = 16
CHUNK = N // NT
LCH = CHUNK // 16
RB = 256
NPASS = 4


def _sc_sort(code):
    mesh = plsc.VectorSubcoreMesh(core_axis_name="c", subcore_axis_name="s")

    @functools.partial(
        pl.kernel, mesh=mesh,
        compiler_params=pltpu.CompilerParams(needs_layout_passes=False,
                                             use_tc_tiling_on_sc=False),
        out_type=jax.ShapeDtypeStruct((2, N), jnp.int32),
        scratch_types=[
            pltpu.VMEM((CHUNK,), jnp.int32),
            pltpu.VMEM((CHUNK,), jnp.int32),
            pltpu.VMEM((RB * 16,), jnp.int32),
            pltpu.VMEM((RB,), jnp.int32),
            pltpu.VMEM((RB,), jnp.int32),
            pltpu.VMEM((NT, RB), jnp.int32),
            pltpu.VMEM((CHUNK,), jnp.int32),
            pltpu.VMEM((CHUNK,), jnp.int32),
            pltpu.VMEM((CHUNK,), jnp.int32),
            pltpu.VMEM_SHARED((N,), jnp.int32),
            pltpu.VMEM_SHARED((N,), jnp.int32),
            pltpu.VMEM_SHARED((N,), jnp.int32),
            pltpu.VMEM_SHARED((N,), jnp.int32),
            pltpu.VMEM_SHARED((NT, RB), jnp.int32),
        ],
    )
    def k(code_h, order_h, mykeys, myvals, hist, dbase, tilecnt,
          allcnt, keybuf, valbuf, destbuf, skA, svA, skB, svB, scnt):
        c = lax.axis_index("c")
        t = lax.axis_index("s")
        lane = lax.iota(jnp.int32, 16)
        ones = jnp.ones((16,), jnp.int32)
        zeros = jnp.zeros((16,), jnp.int32)
        base_t = t * CHUNK

        bufs = [(skB, svB), (skA, svA)]
        for p in range(NPASS):
            shift = 8 * p
            dst_k, dst_v = bufs[p % 2]
            if p == 0:
                pltpu.sync_copy(code_h.at[c, pl.ds(base_t, CHUNK)], mykeys)
            else:
                src_k, src_v = bufs[(p + 1) % 2]
                pltpu.sync_copy(src_k.at[pl.ds(base_t, CHUNK)], mykeys)
                pltpu.sync_copy(src_v.at[pl.ds(base_t, CHUNK)], myvals)

            def zbody(i, _):
                plsc.store_scatter(hist, [i * 16 + lane], zeros)
                return 0
            lax.fori_loop(0, RB, zbody, 0)

            def hbody(i, _):
                kv = plsc.load_gather(mykeys, [lane * LCH + i])
                d = (kv >> shift) & (RB - 1)
                plsc.addupdate_scatter(hist, [d * 16 + lane], ones)
                return 0
            lax.fori_loop(0, LCH, hbody, 0)

            def b1(d, _):
                cell = d * 16 + lane
                row = plsc.load_gather(hist, [cell])
                cs = plsc.cumsum(row)
                plsc.store_scatter(hist, [cell], cs - row)
                plsc.store_scatter(tilecnt, [zeros + d], cs, mask=lane == 15)
                return 0
            lax.fori_loop(0, RB, b1, 0)

            pltpu.sync_copy(tilecnt, scnt.at[t])
            plsc.subcore_barrier()
            pltpu.sync_copy(scnt, allcnt)

            carry = jnp.int32(0)
            for dg in range(RB // 16):
                acc = zeros
                myexcl = zeros
                for tt in range(NT):
                    myexcl = jnp.where(t == tt, acc, myexcl)
                    acc = acc + allcnt[tt, dg * 16:(dg + 1) * 16]
                cs = plsc.cumsum(acc)
                dbase[dg * 16:(dg + 1) * 16] = cs - acc + carry + myexcl
                carry = carry + jnp.sum(acc)

            def cbody(i, _):
                idx = lane * LCH + i
                kv = plsc.load_gather(mykeys, [idx])
                if p == 0:
                    vv = base_t + idx
                else:
                    vv = plsc.load_gather(myvals, [idx])
                d = (kv >> shift) & (RB - 1)
                cell = d * 16 + lane
                cnt = plsc.load_gather(hist, [cell])
                plsc.store_scatter(hist, [cell], cnt + 1)
                db = plsc.load_gather(dbase, [d])
                st = i * 16 + lane
                plsc.store_scatter(keybuf, [st], kv)
                plsc.store_scatter(valbuf, [st], vv)
                plsc.store_scatter(destbuf, [st], db + cnt)
                return 0
            lax.fori_loop(0, LCH, cbody, 0)

            pltpu.sync_copy(keybuf, dst_k.at[destbuf])
            pltpu.sync_copy(valbuf, dst_v.at[destbuf])
            plsc.subcore_barrier()

        _, fin_v = bufs[(NPASS - 1) % 2]
        pltpu.sync_copy(fin_v.at[pl.ds(base_t, CHUNK)], myvals)
        pltpu.sync_copy(myvals, order_h.at[c, pl.ds(base_t, CHUNK)])

    return k(code)


RCH = N // 32
SUB = 512


def _sc_scatter_rows(o, order, row):
    mesh = plsc.VectorSubcoreMesh(core_axis_name="c", subcore_axis_name="s")

    @functools.partial(
        pl.kernel, mesh=mesh,
        compiler_params=pltpu.CompilerParams(needs_layout_passes=False,
                                             use_tc_tiling_on_sc=False),
        out_type=jax.ShapeDtypeStruct((N, DR), jnp.float32),
        scratch_types=[
            pltpu.VMEM((RCH // SUB, SUB), jnp.int32),
            pltpu.VMEM((SUB, DR), jnp.float32),
            pltpu.SemaphoreType.DMA,
        ],
    )
    def k(o_h, ord_h, t_h, idxbuf, obuf, sem):
        c = lax.axis_index("c")
        s = lax.axis_index("s")
        wid = s * 2 + c
        for j in range(RCH // SUB):
            base = wid * RCH + j * SUB
            pltpu.sync_copy(ord_h.at[row, pl.ds(base, SUB)], idxbuf.at[j])
            pltpu.sync_copy(o_h.at[pl.ds(base, SUB), :], obuf)
            pltpu.async_copy(obuf, t_h.at[idxbuf.at[j]], sem).wait()

    return k(o, order)


def _sc_gather_rows(x, t, order, row, with_add):
    mesh = plsc.VectorSubcoreMesh(core_axis_name="c", subcore_axis_name="s")

    @functools.partial(
        pl.kernel, mesh=mesh,
        compiler_params=pltpu.CompilerParams(needs_layout_passes=False,
                                             use_tc_tiling_on_sc=False),
        out_type=jax.ShapeDtypeStruct((N, DR), jnp.float32),
        scratch_types=[
            pltpu.VMEM((RCH // SUB, SUB), jnp.int32),
            pltpu.VMEM((SUB, DR), jnp.float32),
            pltpu.SemaphoreType.DMA,
        ],
    )
    def k(x_h, t_h, ord_h, out_h, idxbuf, obuf, sem):
        c = lax.axis_index("c")
        s = lax.axis_index("s")
        wid = s * 2 + c
        for j in range(RCH // SUB):
            base = wid * RCH + j * SUB
            pltpu.sync_copy(ord_h.at[row, pl.ds(base, SUB)], idxbuf.at[j])
            pltpu.async_copy(x_h.at[idxbuf.at[j]], obuf, sem).wait()
            if with_add:
                pltpu.async_copy(t_h.at[idxbuf.at[j]], obuf, sem, add=True).wait()
            pltpu.sync_copy(obuf, out_h.at[pl.ds(base, SUB), :])

    return k(x, t, order)


def _embed_body(feat_ref, we_ref, be_ref, x_ref):
    x_ref[:, 0:D] = jnp.dot(feat_ref[...], we_ref[...],
                            preferred_element_type=jnp.float32) + be_ref[...]


def _embed(feat, we, be):
    blk = 4096
    return pl.pallas_call(
        _embed_body,
        grid=(N // blk,),
        in_specs=[
            pl.BlockSpec((blk, D_IN), lambda i: (i, 0)),
            pl.BlockSpec((D_IN, D), lambda i: (0, 0)),
            pl.BlockSpec((1, D), lambda i: (0, 0)),
        ],
        out_specs=pl.BlockSpec((blk, DR), lambda i: (i, 0)),
        out_shape=jax.ShapeDtypeStruct((N, DR), jnp.float32),
    )(feat, we, be)


def _attn_body(g_ref, wqkv_ref, wo_ref, o_ref):
    x = g_ref[:, 0:D]
    qkvs = [jnp.dot(x[p * PATCH:(p + 1) * PATCH], wqkv_ref[...],
                    preferred_element_type=jnp.float32) for p in range(PB)]
    ss = [jax.lax.dot_general(
              qkvs[p][:, h * DH:(h + 1) * DH],
              qkvs[p][:, D + h * DH:D + (h + 1) * DH],
              (((1,), (1,)), ((), ())), preferred_element_type=jnp.float32)
          for p in range(PB) for h in range(H)]
    ms = [jnp.max(s, axis=-1, keepdims=True) for s in ss]
    es = [jnp.exp(s - m) for s, m in zip(ss, ms)]
    dens = [jnp.sum(e, axis=-1, keepdims=True) for e in es]
    aas = [e / den for e, den in zip(es, dens)]
    avs = [jnp.dot(aas[p * H + h],
                   qkvs[p][:, 2 * D + h * DH:2 * D + (h + 1) * DH],
                   preferred_element_type=jnp.float32)
           for p in range(PB) for h in range(H)]
    for p in range(PB):
        o = jnp.concatenate(avs[p * H:(p + 1) * H], axis=1)
        o_ref[p * PATCH:(p + 1) * PATCH, 0:D] = jnp.dot(
            o, wo_ref[...], preferred_element_type=jnp.float32)


def _attn_pass(g, wqkv, wo):
    blk = PB * PATCH
    return pl.pallas_call(
        _attn_body,
        grid=(N // blk,),
        in_specs=[
            pl.BlockSpec((blk, DR), lambda i: (i, 0)),
            pl.BlockSpec((D, 3 * D), lambda i: (0, 0)),
            pl.BlockSpec((D, D), lambda i: (0, 0)),
        ],
        out_specs=pl.BlockSpec((blk, DR), lambda i: (i, 0)),
        out_shape=jax.ShapeDtypeStruct((N, DR), jnp.float32),
    )(g, wqkv, wo)


def _head_body(x_ref, t_ref, t2_ref, wh_ref, bh_ref, probs_ref, label_ref):
    x = x_ref[:, 0:D] + t_ref[:, 0:D] + t2_ref[:, 0:D]
    logits = jnp.dot(x, wh_ref[...], preferred_element_type=jnp.float32) + bh_ref[...]
    m = jnp.max(logits, axis=-1, keepdims=True)
    e = jnp.exp(logits - m)
    probs = e / jnp.sum(e, axis=-1, keepdims=True)
    probs_ref[...] = probs
    label_ref[...] = jnp.argmax(logits, axis=-1).astype(jnp.int32)


def _head(x, t, t2, wh, bh):
    blk = 4096
    return pl.pallas_call(
        _head_body,
        grid=(N // blk,),
        in_specs=[
            pl.BlockSpec((blk, DR), lambda i: (i, 0)),
            pl.BlockSpec((blk, DR), lambda i: (i, 0)),
            pl.BlockSpec((blk, DR), lambda i: (i, 0)),
            pl.BlockSpec((D, C), lambda i: (0, 0)),
            pl.BlockSpec((1, C), lambda i: (0, 0)),
        ],
        out_specs=[
            pl.BlockSpec((blk, C), lambda i: (i, 0)),
            pl.BlockSpec((blk,), lambda i: (i,)),
        ],
        out_shape=[
            jax.ShapeDtypeStruct((N, C), jnp.float32),
            jax.ShapeDtypeStruct((N,), jnp.int32),
        ],
    )(x, t, t2, wh, bh)


def kernel(grid_coord, feat, serialized_depth, serialized_code, W_embed,
           b_embed, W_qkv, W_o, W_head, b_head):
    feat = feat.astype(jnp.float32)
    code = serialized_code.astype(jnp.int32)
    be = b_embed.reshape(1, D)
    bh = b_head.reshape(1, C)
    W_qkv = W_qkv.at[:, :, :D].multiply(0.25)

    order = _sc_sort(code)
    x = _embed(feat, W_embed, be)

    g0 = _sc_gather_rows(x, x, order, 0, with_add=False)
    o0 = _attn_pass(g0, W_qkv[0], W_o[0])
    t = _sc_scatter_rows(o0, order, 0)

    g1 = _sc_gather_rows(x, t, order, 1, with_add=True)
    o1 = _attn_pass(g1, W_qkv[1], W_o[1])
    t2 = _sc_scatter_rows(o1, order, 1)

    probs, label = _head(x, t, t2, W_head, bh)
    return (label, probs)


# speedup vs baseline: 3.4519x; 1.0689x over previous
import functools

import jax
import jax.numpy as jnp
from jax import lax
from jax.experimental import pallas as pl
from jax.experimental.pallas import tpu as pltpu
from jax.experimental.pallas import tpu_sc as plsc

N = 65536
D_IN = 6
D = 64
DR = 128
H = 4
DH = D // H
PATCH = 256
C = 19
PB = 16

NT = 16
CHUNK = N // NT
LCH = CHUNK // 16
RB = 256
NPASS = 4


def _sc_sort(code):
    mesh = plsc.VectorSubcoreMesh(core_axis_name="c", subcore_axis_name="s")

    @functools.partial(
        pl.kernel, mesh=mesh,
        compiler_params=pltpu.CompilerParams(needs_layout_passes=False,
                                             use_tc_tiling_on_sc=False),
        out_type=jax.ShapeDtypeStruct((2, N), jnp.int32),
        scratch_types=[
            pltpu.VMEM((CHUNK,), jnp.int32),
            pltpu.VMEM((CHUNK,), jnp.int32),
            pltpu.VMEM((RB * 16,), jnp.int32),
            pltpu.VMEM((RB,), jnp.int32),
            pltpu.VMEM((RB,), jnp.int32),
            pltpu.VMEM((NT, RB), jnp.int32),
            pltpu.VMEM((CHUNK,), jnp.int32),
            pltpu.VMEM((CHUNK,), jnp.int32),
            pltpu.VMEM((CHUNK,), jnp.int32),
            pltpu.VMEM_SHARED((N,), jnp.int32),
            pltpu.VMEM_SHARED((N,), jnp.int32),
            pltpu.VMEM_SHARED((N,), jnp.int32),
            pltpu.VMEM_SHARED((N,), jnp.int32),
            pltpu.VMEM_SHARED((NT, RB), jnp.int32),
        ],
    )
    def k(code_h, order_h, mykeys, myvals, hist, dbase, tilecnt,
          allcnt, keybuf, valbuf, destbuf, skA, svA, skB, svB, scnt):
        c = lax.axis_index("c")
        t = lax.axis_index("s")
        lane = lax.iota(jnp.int32, 16)
        ones = jnp.ones((16,), jnp.int32)
        zeros = jnp.zeros((16,), jnp.int32)
        base_t = t * CHUNK

        bufs = [(skB, svB), (skA, svA)]
        for p in range(NPASS):
            shift = 8 * p
            dst_k, dst_v = bufs[p % 2]
            if p == 0:
                pltpu.sync_copy(code_h.at[c, pl.ds(base_t, CHUNK)], mykeys)
            else:
                src_k, src_v = bufs[(p + 1) % 2]
                pltpu.sync_copy(src_k.at[pl.ds(base_t, CHUNK)], mykeys)
                pltpu.sync_copy(src_v.at[pl.ds(base_t, CHUNK)], myvals)

            def zbody(i, _):
                plsc.store_scatter(hist, [i * 16 + lane], zeros)
                return 0
            lax.fori_loop(0, RB, zbody, 0)

            def hbody(i, _):
                kv = plsc.load_gather(mykeys, [lane * LCH + i])
                d = (kv >> shift) & (RB - 1)
                plsc.addupdate_scatter(hist, [d * 16 + lane], ones)
                return 0
            lax.fori_loop(0, LCH, hbody, 0)

            def b1(d, _):
                cell = d * 16 + lane
                row = plsc.load_gather(hist, [cell])
                cs = plsc.cumsum(row)
                plsc.store_scatter(hist, [cell], cs - row)
                plsc.store_scatter(tilecnt, [zeros + d], cs, mask=lane == 15)
                return 0
            lax.fori_loop(0, RB, b1, 0)

            pltpu.sync_copy(tilecnt, scnt.at[t])
            plsc.subcore_barrier()
            pltpu.sync_copy(scnt, allcnt)

            carry = jnp.int32(0)
            for dg in range(RB // 16):
                acc = zeros
                myexcl = zeros
                for tt in range(NT):
                    myexcl = jnp.where(t == tt, acc, myexcl)
                    acc = acc + allcnt[tt, dg * 16:(dg + 1) * 16]
                cs = plsc.cumsum(acc)
                dbase[dg * 16:(dg + 1) * 16] = cs - acc + carry + myexcl
                carry = carry + jnp.sum(acc)

            def cbody(i, _):
                idx = lane * LCH + i
                kv = plsc.load_gather(mykeys, [idx])
                if p == 0:
                    vv = base_t + idx
                else:
                    vv = plsc.load_gather(myvals, [idx])
                d = (kv >> shift) & (RB - 1)
                cell = d * 16 + lane
                cnt = plsc.load_gather(hist, [cell])
                plsc.store_scatter(hist, [cell], cnt + 1)
                db = plsc.load_gather(dbase, [d])
                st = i * 16 + lane
                plsc.store_scatter(keybuf, [st], kv)
                plsc.store_scatter(valbuf, [st], vv)
                plsc.store_scatter(destbuf, [st], db + cnt)
                return 0
            lax.fori_loop(0, LCH, cbody, 0)

            pltpu.sync_copy(keybuf, dst_k.at[destbuf])
            pltpu.sync_copy(valbuf, dst_v.at[destbuf])
            plsc.subcore_barrier()

        _, fin_v = bufs[(NPASS - 1) % 2]
        pltpu.sync_copy(fin_v.at[pl.ds(base_t, CHUNK)], myvals)
        pltpu.sync_copy(myvals, order_h.at[c, pl.ds(base_t, CHUNK)])

    return k(code)


RCH = N // 32
SUB = 512


def _sc_scatter_rows(o, order, row):
    mesh = plsc.VectorSubcoreMesh(core_axis_name="c", subcore_axis_name="s")

    @functools.partial(
        pl.kernel, mesh=mesh,
        compiler_params=pltpu.CompilerParams(needs_layout_passes=False,
                                             use_tc_tiling_on_sc=False),
        out_type=jax.ShapeDtypeStruct((N, DR), jnp.float32),
        scratch_types=[
            pltpu.VMEM((RCH // SUB, SUB), jnp.int32),
            pltpu.VMEM((SUB, DR), jnp.float32),
            pltpu.SemaphoreType.DMA,
        ],
    )
    def k(o_h, ord_h, t_h, idxbuf, obuf, sem):
        c = lax.axis_index("c")
        s = lax.axis_index("s")
        wid = s * 2 + c
        for j in range(RCH // SUB):
            base = wid * RCH + j * SUB
            pltpu.sync_copy(ord_h.at[row, pl.ds(base, SUB)], idxbuf.at[j])
            pltpu.sync_copy(o_h.at[pl.ds(base, SUB), :], obuf)
            pltpu.async_copy(obuf, t_h.at[idxbuf.at[j]], sem).wait()

    return k(o, order)


def _sc_gather_rows(x, t, order, row, with_add):
    mesh = plsc.VectorSubcoreMesh(core_axis_name="c", subcore_axis_name="s")

    @functools.partial(
        pl.kernel, mesh=mesh,
        compiler_params=pltpu.CompilerParams(needs_layout_passes=False,
                                             use_tc_tiling_on_sc=False),
        out_type=jax.ShapeDtypeStruct((N, DR), jnp.float32),
        scratch_types=[
            pltpu.VMEM((RCH // SUB, SUB), jnp.int32),
            pltpu.VMEM((SUB, DR), jnp.float32),
            pltpu.SemaphoreType.DMA,
        ],
    )
    def k(x_h, t_h, ord_h, out_h, idxbuf, obuf, sem):
        c = lax.axis_index("c")
        s = lax.axis_index("s")
        wid = s * 2 + c
        for j in range(RCH // SUB):
            base = wid * RCH + j * SUB
            pltpu.sync_copy(ord_h.at[row, pl.ds(base, SUB)], idxbuf.at[j])
            pltpu.async_copy(x_h.at[idxbuf.at[j]], obuf, sem).wait()
            if with_add:
                pltpu.async_copy(t_h.at[idxbuf.at[j]], obuf, sem, add=True).wait()
            pltpu.sync_copy(obuf, out_h.at[pl.ds(base, SUB), :])

    return k(x, t, order)


def _embed_body(feat_ref, we_ref, be_ref, x_ref):
    x_ref[:, 0:D] = jnp.dot(feat_ref[...], we_ref[...],
                            preferred_element_type=jnp.float32) + be_ref[...]


def _embed(feat, we, be):
    blk = 4096
    return pl.pallas_call(
        _embed_body,
        grid=(N // blk,),
        in_specs=[
            pl.BlockSpec((blk, D_IN), lambda i: (i, 0)),
            pl.BlockSpec((D_IN, D), lambda i: (0, 0)),
            pl.BlockSpec((1, D), lambda i: (0, 0)),
        ],
        out_specs=pl.BlockSpec((blk, DR), lambda i: (i, 0)),
        out_shape=jax.ShapeDtypeStruct((N, DR), jnp.float32),
    )(feat, we, be)


def _attn_body(g_ref, wqkv_ref, wo_ref, o_ref):
    x = g_ref[:, 0:D]
    wqv = jnp.concatenate([wqkv_ref[:, 0:D] * 0.25, wqkv_ref[:, 2 * D:]], axis=1)
    wkT = wqkv_ref[:, D:2 * D].T
    qkvs = [jnp.dot(x[p * PATCH:(p + 1) * PATCH], wqv,
                    preferred_element_type=jnp.float32) for p in range(PB)]
    kTs = [jnp.dot(wkT, x[p * PATCH:(p + 1) * PATCH].T,
                   preferred_element_type=jnp.float32) for p in range(PB)]
    ss = [jnp.dot(qkvs[p][:, h * DH:(h + 1) * DH],
                  kTs[p][h * DH:(h + 1) * DH, :],
                  preferred_element_type=jnp.float32)
          for p in range(PB) for h in range(H)]
    ms = [jnp.max(s, axis=-1, keepdims=True) for s in ss]
    es = [jnp.exp(s - m) for s, m in zip(ss, ms)]
    dens = [jnp.sum(e, axis=-1, keepdims=True) for e in es]
    aas = [e / den for e, den in zip(es, dens)]
    avs = [jnp.dot(aas[p * H + h],
                   qkvs[p][:, D + h * DH:D + (h + 1) * DH],
                   preferred_element_type=jnp.float32)
           for p in range(PB) for h in range(H)]
    for p in range(PB):
        o = jnp.concatenate(avs[p * H:(p + 1) * H], axis=1)
        o_ref[p * PATCH:(p + 1) * PATCH, 0:D] = jnp.dot(
            o, wo_ref[...], preferred_element_type=jnp.float32)


def _attn_pass(g, wqkv, wo):
    blk = PB * PATCH
    return pl.pallas_call(
        _attn_body,
        grid=(N // blk,),
        in_specs=[
            pl.BlockSpec((blk, DR), lambda i: (i, 0)),
            pl.BlockSpec((D, 3 * D), lambda i: (0, 0)),
            pl.BlockSpec((D, D), lambda i: (0, 0)),
        ],
        out_specs=pl.BlockSpec((blk, DR), lambda i: (i, 0)),
        out_shape=jax.ShapeDtypeStruct((N, DR), jnp.float32),
    )(g, wqkv, wo)


def _head_body(x_ref, t_ref, t2_ref, wh_ref, bh_ref, probs_ref, label_ref):
    x = x_ref[:, 0:D] + t_ref[:, 0:D] + t2_ref[:, 0:D]
    logits = jnp.dot(x, wh_ref[...], preferred_element_type=jnp.float32) + bh_ref[...]
    m = jnp.max(logits, axis=-1, keepdims=True)
    e = jnp.exp(logits - m)
    probs = e / jnp.sum(e, axis=-1, keepdims=True)
    probs_ref[...] = probs
    label_ref[...] = jnp.argmax(logits, axis=-1).astype(jnp.int32)


def _head(x, t, t2, wh, bh):
    blk = 4096
    return pl.pallas_call(
        _head_body,
        grid=(N // blk,),
        in_specs=[
            pl.BlockSpec((blk, DR), lambda i: (i, 0)),
            pl.BlockSpec((blk, DR), lambda i: (i, 0)),
            pl.BlockSpec((blk, DR), lambda i: (i, 0)),
            pl.BlockSpec((D, C), lambda i: (0, 0)),
            pl.BlockSpec((1, C), lambda i: (0, 0)),
        ],
        out_specs=[
            pl.BlockSpec((blk, C), lambda i: (i, 0)),
            pl.BlockSpec((blk,), lambda i: (i,)),
        ],
        out_shape=[
            jax.ShapeDtypeStruct((N, C), jnp.float32),
            jax.ShapeDtypeStruct((N,), jnp.int32),
        ],
    )(x, t, t2, wh, bh)


def kernel(grid_coord, feat, serialized_depth, serialized_code, W_embed,
           b_embed, W_qkv, W_o, W_head, b_head):
    feat = feat.astype(jnp.float32)
    code = serialized_code.astype(jnp.int32)
    be = b_embed.reshape(1, D)
    bh = b_head.reshape(1, C)
    order = _sc_sort(code)
    x = _embed(feat, W_embed, be)

    g0 = _sc_gather_rows(x, x, order, 0, with_add=False)
    o0 = _attn_pass(g0, W_qkv[0], W_o[0])
    t = _sc_scatter_rows(o0, order, 0)

    g1 = _sc_gather_rows(x, t, order, 1, with_add=True)
    o1 = _attn_pass(g1, W_qkv[1], W_o[1])
    t2 = _sc_scatter_rows(o1, order, 1)

    probs, label = _head(x, t, t2, W_head, bh)
    return (label, probs)
